# baseline (speedup 1.0000x reference)
import functools

import jax
import jax.numpy as jnp
from jax import lax
from jax.experimental import pallas as pl
from jax.experimental.pallas import tpu as pltpu
from jax.experimental.pallas import tpu_sc as plsc

B = 4096
L = 200
D = 64
N_TOTAL = B * L
NC, NS = 2, 16
NW = NC * NS
PER_W = N_TOTAL // NW
G = 128
N_GROUPS = PER_W // G

_mesh = plsc.VectorSubcoreMesh(core_axis_name="c", subcore_axis_name="s")


@functools.partial(
    pl.kernel,
    out_type=jax.ShapeDtypeStruct((N_TOTAL, D), jnp.float32),
    mesh=_mesh,
    scratch_types=[
        pltpu.VMEM((G,), jnp.int32),
        pltpu.VMEM((G,), jnp.int32),
        pltpu.VMEM((G, D), jnp.float32),
        pltpu.SemaphoreType.DMA,
    ],
    compiler_params=pltpu.CompilerParams(use_tc_tiling_on_sc=False),
)
def _embed(idx_hbm, role_hbm, tok_hbm, rt_hbm, out_hbm, idx_v, role_v, rows_v, sem):
    wid = lax.axis_index("s") * NC + lax.axis_index("c")
    base_w = wid * PER_W

    @pl.loop(0, N_GROUPS)
    def _group(g):
        base = base_w + g * G
        pltpu.sync_copy(idx_hbm.at[pl.ds(base, G)], idx_v)
        pltpu.sync_copy(role_hbm.at[pl.ds(base, G)], role_v)
        pltpu.async_copy(rt_hbm.at[role_v], rows_v, sem).wait()
        pltpu.async_copy(tok_hbm.at[idx_v], rows_v, sem, add=True).wait()
        pltpu.sync_copy(rows_v, out_hbm.at[pl.ds(base, G)])


def kernel(idx, role_ids, token_table, role_table):
    flat_idx = idx.reshape(-1).astype(jnp.int32)
    flat_role = role_ids.reshape(-1).astype(jnp.int32)
    out = _embed(flat_idx, flat_role, token_table, role_table)
    return out.reshape(B, L, D)


# speedup vs baseline: 1.0041x; 1.0041x over previous
import functools

import jax
import jax.numpy as jnp
from jax import lax
from jax.experimental import pallas as pl
from jax.experimental.pallas import tpu as pltpu
from jax.experimental.pallas import tpu_sc as plsc

B = 4096
L = 200
D = 64
N_TOTAL = B * L
NC, NS = 2, 16
NW = NC * NS
PER_W = N_TOTAL // NW
G = 128
K = 10
N_SUPER = PER_W // (G * K)

_mesh = plsc.VectorSubcoreMesh(core_axis_name="c", subcore_axis_name="s")


@functools.partial(
    pl.kernel,
    out_type=jax.ShapeDtypeStruct((N_TOTAL, D), jnp.float32),
    mesh=_mesh,
    scratch_types=[
        pltpu.VMEM((K, G), jnp.int32),
        pltpu.VMEM((K, G), jnp.int32),
        pltpu.VMEM((K, G, D), jnp.float32),
        pltpu.SemaphoreType.DMA,
        pltpu.SemaphoreType.DMA,
    ],
    compiler_params=pltpu.CompilerParams(use_tc_tiling_on_sc=False),
)
def _embed(idx_hbm, role_hbm, tok_hbm, rt_hbm, out_hbm, idx_v, role_v, rows_v, sem_a, sem_b):
    wid = lax.axis_index("s") * NC + lax.axis_index("c")
    base_w = wid * PER_W

    @pl.loop(0, N_SUPER)
    def _super(s):
        base = base_w + s * (G * K)
        grp = base_w // G + s * K
        idx_cp = pltpu.async_copy(idx_hbm.at[pl.ds(grp, K)], idx_v, sem_a)
        role_cp = pltpu.async_copy(role_hbm.at[pl.ds(grp, K)], role_v, sem_b)
        idx_cp.wait()
        role_cp.wait()
        cps = [pltpu.async_copy(rt_hbm.at[role_v.at[b]], rows_v.at[b], sem_a)
               for b in range(K)]
        for cp in cps:
            cp.wait()
        cps = [pltpu.async_copy(tok_hbm.at[idx_v.at[b]], rows_v.at[b], sem_b,
                                add=True)
               for b in range(K)]
        for cp in cps:
            cp.wait()
        cps = [pltpu.async_copy(rows_v.at[b],
                                out_hbm.at[pl.ds(base + b * G, G)], sem_a)
               for b in range(K)]
        for cp in cps:
            cp.wait()


def kernel(idx, role_ids, token_table, role_table):
    flat_idx = idx.reshape(N_TOTAL // G, G).astype(jnp.int32)
    flat_role = role_ids.reshape(N_TOTAL // G, G).astype(jnp.int32)
    out = _embed(flat_idx, flat_role, token_table, role_table)
    return out.reshape(B, L, D)


# speedup vs baseline: 8.7952x; 8.7589x over previous
import functools

import jax
import jax.numpy as jnp
from jax import lax
from jax.experimental import pallas as pl
from jax.experimental.pallas import tpu as pltpu
from jax.experimental.pallas import tpu_sc as plsc

B = 4096
L = 200
D = 64
N_TOTAL = B * L
NC, NS = 2, 16
NW = NC * NS
PER_W = N_TOTAL // NW
G = 128
K = 10
N_SUPER = PER_W // (G * K)

_mesh = plsc.VectorSubcoreMesh(core_axis_name="c", subcore_axis_name="s")


@functools.partial(
    pl.kernel,
    out_type=jax.ShapeDtypeStruct((N_TOTAL, D), jnp.float32),
    mesh=_mesh,
    scratch_types=[
        pltpu.VMEM((K, G), jnp.int32),
        pltpu.VMEM((K, G), jnp.int32),
        pltpu.VMEM((K, G, D), jnp.float32),
        pltpu.VMEM_SHARED((3, D), jnp.float32),
        pltpu.SemaphoreType.DMA,
        pltpu.SemaphoreType.DMA,
    ],
    compiler_params=pltpu.CompilerParams(use_tc_tiling_on_sc=False),
)
def _embed(idx_hbm, role_hbm, tok_hbm, rt_hbm, out_hbm, idx_v, role_v, rows_v,
           rt_v, sem_a, sem_b):
    wid = lax.axis_index("s") * NC + lax.axis_index("c")
    base_w = wid * PER_W
    pltpu.sync_copy(rt_hbm, rt_v)

    @pl.loop(0, N_SUPER)
    def _super(s):
        base = base_w + s * (G * K)
        grp = base_w // G + s * K
        idx_cp = pltpu.async_copy(idx_hbm.at[pl.ds(grp, K)], idx_v, sem_a)
        role_cp = pltpu.async_copy(role_hbm.at[pl.ds(grp, K)], role_v, sem_b)
        idx_cp.wait()
        role_cp.wait()
        cps = [pltpu.async_copy(rt_v.at[role_v.at[b]], rows_v.at[b], sem_a)
               for b in range(K)]
        for cp in cps:
            cp.wait()
        cps = [pltpu.async_copy(tok_hbm.at[idx_v.at[b]], rows_v.at[b], sem_b,
                                add=True)
               for b in range(K)]
        for cp in cps:
            cp.wait()
        cps = [pltpu.async_copy(rows_v.at[b],
                                out_hbm.at[pl.ds(base + b * G, G)], sem_a)
               for b in range(K)]
        for cp in cps:
            cp.wait()


def kernel(idx, role_ids, token_table, role_table):
    flat_idx = idx.reshape(N_TOTAL // G, G).astype(jnp.int32)
    flat_role = role_ids.reshape(N_TOTAL // G, G).astype(jnp.int32)
    out = _embed(flat_idx, flat_role, token_table, role_table)
    return out.reshape(B, L, D)


# speedup vs baseline: 9.0477x; 1.0287x over previous
import functools

import jax
import jax.numpy as jnp
from jax import lax
from jax.experimental import pallas as pl
from jax.experimental.pallas import tpu as pltpu
from jax.experimental.pallas import tpu_sc as plsc

B = 4096
L = 200
D = 64
N_TOTAL = B * L
NC, NS = 2, 16
NW = NC * NS
PER_W = N_TOTAL // NW
G = 128
SET = 5
N_SUPER = PER_W // (G * SET)

_mesh = plsc.VectorSubcoreMesh(core_axis_name="c", subcore_axis_name="s")


@functools.partial(
    pl.kernel,
    out_type=jax.ShapeDtypeStruct((N_TOTAL, D), jnp.float32),
    mesh=_mesh,
    scratch_types=[
        pltpu.VMEM((2, SET, 2, G), jnp.int32),
        pltpu.VMEM((2, SET, G, D), jnp.float32),
        pltpu.VMEM_SHARED((3, D), jnp.float32),
        pltpu.SemaphoreType.DMA,
        pltpu.SemaphoreType.DMA,
        pltpu.SemaphoreType.DMA,
        pltpu.SemaphoreType.DMA,
        pltpu.SemaphoreType.DMA,
        pltpu.SemaphoreType.DMA,
    ],
    compiler_params=pltpu.CompilerParams(use_tc_tiling_on_sc=False),
)
def _embed(pairs_hbm, tok_hbm, rt_hbm, out_hbm, idx_v, rows_v, rt_sh,
           sem_i0, sem_i1, sem_role, sem_tok, sem_w0, sem_w1):
    wid = lax.axis_index("s") * NC + lax.axis_index("c")
    base_w = wid * PER_W
    grp_w = base_w // G
    pltpu.sync_copy(rt_hbm, rt_sh)

    sem_i = (sem_i0, sem_i1)
    sem_w = (sem_w0, sem_w1)

    def idx_copy(s, p):
        return pltpu.async_copy(
            pairs_hbm.at[pl.ds(grp_w + s * SET, SET)], idx_v.at[p], sem_i[p])

    idx_copy(0, 0)

    @pl.loop(0, N_SUPER, step=2)
    def _super(s0):
        for p in range(2):
            s = s0 + p
            q = 1 - p
            base = base_w + s * (SET * G)
            @pl.when(s >= 2)
            def _():
                for b in range(SET):
                    pltpu.make_async_copy(
                        rows_v.at[p, b],
                        out_hbm.at[pl.ds(base, G)], sem_w[p]).wait()
            @pl.when(s + 1 < N_SUPER)
            def _():
                idx_copy(s + 1, q)
            pltpu.make_async_copy(
                pairs_hbm.at[pl.ds(0, SET)], idx_v.at[p], sem_i[p]).wait()
            cps = [pltpu.async_copy(rt_sh.at[idx_v.at[p, b, 1]],
                                    rows_v.at[p, b], sem_role)
                   for b in range(SET)]
            for cp in cps:
                cp.wait()
            cps = [pltpu.async_copy(tok_hbm.at[idx_v.at[p, b, 0]],
                                    rows_v.at[p, b], sem_tok, add=True)
                   for b in range(SET)]
            for cp in cps:
                cp.wait()
            for b in range(SET):
                pltpu.async_copy(rows_v.at[p, b],
                                 out_hbm.at[pl.ds(base + b * G, G)], sem_w[p])

    for p in range(2):
        for b in range(SET):
            pltpu.make_async_copy(
                rows_v.at[p, b], out_hbm.at[pl.ds(0, G)], sem_w[p]).wait()


def kernel(idx, role_ids, token_table, role_table):
    idx2d = idx.reshape(N_TOTAL // G, G).astype(jnp.int32)
    role2d = role_ids.reshape(N_TOTAL // G, G).astype(jnp.int32)
    pairs = jnp.stack([idx2d, role2d], axis=1)
    out = _embed(pairs, token_table, role_table)
    return out.reshape(B, L, D)


# speedup vs baseline: 9.3310x; 1.0313x over previous
import functools

import jax
import jax.numpy as jnp
from jax import lax
from jax.experimental import pallas as pl
from jax.experimental.pallas import tpu as pltpu
from jax.experimental.pallas import tpu_sc as plsc

B = 4096
L = 200
D = 64
N_TOTAL = B * L
NC, NS = 2, 16
NW = NC * NS
PER_W = N_TOTAL // NW
G = 128
SET = 5
N_SUPER = PER_W // (G * SET)

_mesh = plsc.VectorSubcoreMesh(core_axis_name="c", subcore_axis_name="s")


@functools.partial(
    pl.kernel,
    out_type=jax.ShapeDtypeStruct((N_TOTAL, D), jnp.float32),
    mesh=_mesh,
    scratch_types=[
        pltpu.VMEM((2, SET, 2, G), jnp.int32),
        pltpu.VMEM((2, SET, G, D), jnp.float32),
        pltpu.VMEM_SHARED((3, D), jnp.float32),
        pltpu.SemaphoreType.DMA,
        pltpu.SemaphoreType.DMA,
        pltpu.SemaphoreType.DMA,
        pltpu.SemaphoreType.DMA,
        pltpu.SemaphoreType.DMA,
        pltpu.SemaphoreType.DMA,
    ],
    compiler_params=pltpu.CompilerParams(use_tc_tiling_on_sc=False),
)
def _embed(pairs_hbm, tok_hbm, rt_hbm, out_hbm, idx_v, rows_v, rt_sh,
           sem_i0, sem_i1, sem_role, sem_tok, sem_w0, sem_w1):
    wid = lax.axis_index("s") * NC + lax.axis_index("c")
    base_w = wid * PER_W
    grp_w = base_w // G
    pltpu.sync_copy(rt_hbm, rt_sh)

    sem_i = (sem_i0, sem_i1)
    sem_w = (sem_w0, sem_w1)

    def idx_copy(s, p):
        pltpu.async_copy(
            pairs_hbm.at[pl.ds(grp_w + s * SET, SET)], idx_v.at[p], sem_i[p])

    def idx_wait(p):
        pltpu.make_async_copy(
            pairs_hbm.at[pl.ds(0, SET)], idx_v.at[p], sem_i[p]).wait()

    def fire_roles(p):
        for b in range(SET):
            pltpu.async_copy(rt_sh.at[idx_v.at[p, b, 1]], rows_v.at[p, b],
                             sem_role)

    def drain_roles(p):
        for b in range(SET):
            pltpu.make_async_copy(rt_sh.at[idx_v.at[p, b, 1]],
                                  rows_v.at[p, b], sem_role).wait()

    def drain_wb(p):
        for b in range(SET):
            pltpu.make_async_copy(rows_v.at[p, b], out_hbm.at[pl.ds(0, G)],
                                  sem_w[p]).wait()

    idx_copy(0, 0)
    idx_wait(0)
    fire_roles(0)
    idx_copy(1, 1)

    @pl.loop(0, N_SUPER, step=2)
    def _super(s0):
        for p in range(2):
            s = s0 + p
            q = 1 - p
            base = base_w + s * (SET * G)
            drain_roles(p)
            cps = [pltpu.async_copy(tok_hbm.at[idx_v.at[p, b, 0]],
                                    rows_v.at[p, b], sem_tok, add=True)
                   for b in range(SET)]

            @pl.when(s >= 1)
            def _():
                drain_wb(q)

            @pl.when(s + 1 < N_SUPER)
            def _():
                idx_wait(q)
                fire_roles(q)

            for cp in cps:
                cp.wait()
            @pl.when(s + 2 < N_SUPER)
            def _():
                idx_copy(s + 2, p)
            for b in range(SET):
                pltpu.async_copy(rows_v.at[p, b],
                                 out_hbm.at[pl.ds(base + b * G, G)], sem_w[p])

    drain_wb((N_SUPER - 1) % 2)


def kernel(idx, role_ids, token_table, role_table):
    idx2d = idx.reshape(N_TOTAL // G, G).astype(jnp.int32)
    role2d = role_ids.reshape(N_TOTAL // G, G).astype(jnp.int32)
    pairs = jnp.stack([idx2d, role2d], axis=1)
    out = _embed(pairs, token_table, role_table)
    return out.reshape(B, L, D)
